# baseline (speedup 1.0000x reference)
import functools

import jax
import jax.numpy as jnp
from jax import lax
from jax.experimental import pallas as pl
from jax.experimental.pallas import tpu as pltpu
from jax.experimental.pallas import tpu_sc as plsc

N = 10000
E = 320000
D = 128
NC = 2
NS = 16
L = 16
NW = NC * NS
CHUNK = 80
NCH_DEG = E // NW // CHUNK
CHUNK_A = 128
NCH_AGG = 157

NPD = 10240
NPDT = NPD // NS

NPH = 5120
NPHA = 5248
WROWS = NPH // NS
ZROWS = 80

_sc_mesh = plsc.VectorSubcoreMesh(core_axis_name="c", subcore_axis_name="s")
_sc_params = pltpu.CompilerParams(needs_layout_passes=False)


@functools.partial(
    pl.kernel,
    out_type=jax.ShapeDtypeStruct((NC, NPD), jnp.float32),
    mesh=_sc_mesh,
    scratch_types=[
        pltpu.VMEM((NCH_DEG, CHUNK), jnp.int32),
        pltpu.VMEM((NPD,), jnp.float32),
        pltpu.VMEM((NS, NPDT), jnp.float32),
        pltpu.VMEM((NPDT,), jnp.float32),
        pltpu.VMEM_SHARED((NS, NPD), jnp.float32),
    ],
    compiler_params=_sc_params,
)
def _deg_sc(dst_hbm, deg_hbm, idx_v, hist_v, red_v, ob_v, sh_v):
    c = lax.axis_index("c")
    s = lax.axis_index("s")
    wid = s * NC + c
    pltpu.sync_copy(dst_hbm.at[wid], idx_v)

    @pl.loop(0, NPD, step=L)
    def _(i):
        hist_v[pl.ds(i, L)] = jnp.zeros((L,), jnp.float32)

    ones = jnp.ones((L,), jnp.float32)

    @pl.loop(0, NCH_DEG)
    def _(i):
        @pl.loop(0, CHUNK, step=L)
        def _(j):
            v = idx_v[i, pl.ds(j, L)]
            plsc.addupdate_scatter(hist_v, [v], ones)

    pltpu.sync_copy(hist_v, sh_v.at[s])
    plsc.subcore_barrier()

    col0 = s * NPDT
    for r in range(NS):
        pltpu.sync_copy(sh_v.at[r, pl.ds(col0, NPDT)], red_v.at[r])

    @pl.loop(0, NPDT, step=L)
    def _(j):
        acc = red_v[0, pl.ds(j, L)]
        for r in range(1, NS):
            acc = acc + red_v[r, pl.ds(j, L)]
        ob_v[pl.ds(j, L)] = acc

    pltpu.sync_copy(ob_v, deg_hbm.at[c, pl.ds(col0, NPDT)])


NBR = 4
NBI = 8
UNROLL = 8


@functools.partial(
    pl.kernel,
    out_type=jax.ShapeDtypeStruct((NC, NPH, D), jnp.float32),
    mesh=_sc_mesh,
    scratch_types=[
        pltpu.VMEM((NBI, 2, CHUNK_A), jnp.int32),
        pltpu.VMEM((NBR, CHUNK_A, D), jnp.float32),
        pltpu.SemaphoreType.DMA((NBI,)),
        pltpu.SemaphoreType.DMA((NBR,)),
        pltpu.SemaphoreType.DMA((NBR,)),
        pltpu.VMEM_SHARED((NPHA, D), jnp.float32),
    ],
    compiler_params=_sc_params,
)
def _agg_sc(edges_hbm, data_hbm, out_hbm,
            idx_v, rows_v, isem, gsem, ssem, acc_sh):
    c = lax.axis_index("c")
    s = lax.axis_index("s")

    @pl.loop(0, ZROWS)
    def _(r):
        @pl.loop(0, D, step=L)
        def _(j):
            rows_v[0, r, pl.ds(j, L)] = jnp.zeros((L,), jnp.float32)

    row0 = s * WROWS
    for b in range(WROWS // ZROWS):
        pltpu.sync_copy(rows_v.at[0, pl.ds(0, ZROWS)],
                        acc_sh.at[pl.ds(row0 + b * ZROWS, ZROWS)])
    plsc.subcore_barrier()

    base = c * NPH
    trash = jnp.full((L,), NPH, jnp.int32) + lax.iota(jnp.int32, L)

    def start_idx(i, bi):
        pltpu.async_copy(edges_hbm.at[s, i], idx_v.at[bi], isem.at[bi])

    def wait_idx(i, bi):
        pltpu.make_async_copy(edges_hbm.at[s, i], idx_v.at[bi],
                              isem.at[bi]).wait()

    def translate(bi):
        for j in range(0, CHUNK_A, L):
            d = idx_v[bi, 1, pl.ds(j, L)]
            local = d - base
            inb = (local >= 0) & (local < NPH)
            idx_v[bi, 1, pl.ds(j, L)] = jnp.where(inb, local, trash)

    def start_g(br, bi):
        pltpu.async_copy(data_hbm.at[idx_v.at[bi, 0]], rows_v.at[br],
                         gsem.at[br])

    def wait_g(br, bi):
        pltpu.make_async_copy(data_hbm.at[idx_v.at[bi, 0]], rows_v.at[br],
                              gsem.at[br]).wait()

    def start_s(br, bi):
        pltpu.async_copy(rows_v.at[br], acc_sh.at[idx_v.at[bi, 1]],
                         ssem.at[br], add=True)

    def wait_s(br, bi):
        pltpu.make_async_copy(rows_v.at[br], acc_sh.at[idx_v.at[bi, 1]],
                              ssem.at[br]).wait()

    def slot(j, r, head=False):
        if not isinstance(j, int) or j + 2 < NCH_AGG:
            b2r, b2i = (r + 2) % NBR, (r + 2) % NBI
            wait_idx(j + 2, b2i)
            translate(b2i)
            if not head:
                wait_s(b2r, b2i)
            start_g(b2r, b2i)
        wait_g(r % NBR, r % NBI)
        start_s(r % NBR, r % NBI)
        if isinstance(j, int):
            if j + 6 < NCH_AGG:
                start_idx(j + 6, (r + 6) % NBI)
        else:
            start_idx(j + 6, (r + 6) % NBI)

    for i in range(6):
        start_idx(i, i)
    wait_idx(0, 0)
    translate(0)
    wait_idx(1, 1)
    translate(1)
    start_g(0, 0)
    start_g(1, 1)

    for j in range(UNROLL):
        slot(j, j, head=(j < 2))

    tail_start = ((NCH_AGG - 7) // UNROLL) * UNROLL

    @pl.loop(1, tail_start // UNROLL)
    def _(k):
        for r in range(UNROLL):
            slot(k * UNROLL + r, r)

    for j in range(tail_start, NCH_AGG):
        slot(j, j % UNROLL)

    wait_s((NCH_AGG - 2) % NBR, (NCH_AGG - 2) % NBI)
    wait_s((NCH_AGG - 1) % NBR, (NCH_AGG - 1) % NBI)

    plsc.subcore_barrier()
    pltpu.sync_copy(acc_sh.at[pl.ds(row0, WROWS)], out_hbm.at[c, pl.ds(row0, WROWS)])


BN = 400
GRID = N // BN


def _pre_body(deg_ref, x_ref, xs_ref, dis_ref):
    deg = deg_ref[0] + deg_ref[1] + 1.0
    dis = lax.rsqrt(deg)
    dis_ref[...] = dis
    xs_ref[...] = x_ref[...] * dis


_pre_tc = pl.pallas_call(
    _pre_body,
    grid=(GRID,),
    in_specs=[
        pl.BlockSpec((2, BN, 1), lambda i: (0, i, 0)),
        pl.BlockSpec((BN, D), lambda i: (i, 0)),
    ],
    out_specs=[
        pl.BlockSpec((BN, D), lambda i: (i, 0)),
        pl.BlockSpec((BN, 1), lambda i: (i, 0)),
    ],
    out_shape=[
        jax.ShapeDtypeStruct((N, D), jnp.float32),
        jax.ShapeDtypeStruct((N, 1), jnp.float32),
    ],
)


def _mm_body(p_ref, xs_ref, dis_ref, w1_ref, b1_ref, w2_ref, t_ref):
    dis = dis_ref[...]
    z = (p_ref[...] + xs_ref[...]) * dis
    h = jnp.dot(z, w1_ref[...], preferred_element_type=jnp.float32) + b1_ref[...]
    h = jnp.maximum(h, 0.0)
    t_ref[...] = jnp.dot(h, w2_ref[...], preferred_element_type=jnp.float32) * dis


_mm_tc = pl.pallas_call(
    _mm_body,
    grid=(GRID,),
    in_specs=[
        pl.BlockSpec((BN, D), lambda i: (i, 0)),
        pl.BlockSpec((BN, D), lambda i: (i, 0)),
        pl.BlockSpec((BN, 1), lambda i: (i, 0)),
        pl.BlockSpec((D, 2 * D), lambda i: (0, 0)),
        pl.BlockSpec((1, 2 * D), lambda i: (0, 0)),
        pl.BlockSpec((2 * D, D), lambda i: (0, 0)),
    ],
    out_specs=pl.BlockSpec((BN, D), lambda i: (i, 0)),
    out_shape=jax.ShapeDtypeStruct((N, D), jnp.float32),
)


def _out_body(p_ref, t_ref, dis_ref, b2_ref, o_ref):
    o_ref[...] = (p_ref[...] + t_ref[...]) * dis_ref[...] + b2_ref[...]


_out_tc = pl.pallas_call(
    _out_body,
    grid=(GRID,),
    in_specs=[
        pl.BlockSpec((BN, D), lambda i: (i, 0)),
        pl.BlockSpec((BN, D), lambda i: (i, 0)),
        pl.BlockSpec((BN, 1), lambda i: (i, 0)),
        pl.BlockSpec((1, D), lambda i: (0, 0)),
    ],
    out_specs=pl.BlockSpec((BN, D), lambda i: (i, 0)),
    out_shape=jax.ShapeDtypeStruct((N, D), jnp.float32),
)


def kernel(x, edge_index, conv1_weight, conv1_bias, conv2_weight, conv2_bias):
    dst = edge_index[1]
    dst_deg3 = dst.reshape(NW, NCH_DEG, CHUNK)
    ept = E // NS
    epa = NCH_AGG * CHUNK_A
    pad = ((0, 0), (0, epa - ept))
    srcp = jnp.pad(edge_index[0].reshape(NS, ept), pad)
    dstp = jnp.pad(dst.reshape(NS, ept), pad, constant_values=N)
    edges3 = jnp.stack([srcp.reshape(NS, NCH_AGG, CHUNK_A),
                        dstp.reshape(NS, NCH_AGG, CHUNK_A)], axis=2)

    deg_parts = _deg_sc(dst_deg3)
    deg2 = deg_parts[:, :N].reshape(2, N, 1)
    xs, dis = _pre_tc(deg2, x)
    p1 = _agg_sc(edges3, xs).reshape(NC * NPH, D)
    t = _mm_tc(p1, xs, dis, conv1_weight,
               conv1_bias.reshape(1, 2 * D), conv2_weight)
    p2 = _agg_sc(edges3, t).reshape(NC * NPH, D)
    out = _out_tc(p2, t, dis, conv2_bias.reshape(1, D))
    return out


# speedup vs baseline: 1.2240x; 1.2240x over previous
import functools

import jax
import jax.numpy as jnp
from jax import lax
from jax.experimental import pallas as pl
from jax.experimental.pallas import tpu as pltpu
from jax.experimental.pallas import tpu_sc as plsc

N = 10000
E = 320000
D = 128
NC = 2
NS = 16
L = 16
NW = NC * NS
CHUNK = 80
NCH_DEG = E // NW // CHUNK
CHUNK_A = 64
NCH_AGG = 313

NPD = 10240
NPDT = NPD // NS

NPH = 5120
NPHA = 5248
WROWS = NPH // NS
ZROWS = 80

_sc_mesh = plsc.VectorSubcoreMesh(core_axis_name="c", subcore_axis_name="s")
_sc_params = pltpu.CompilerParams(needs_layout_passes=False)


@functools.partial(
    pl.kernel,
    out_type=jax.ShapeDtypeStruct((NC, NPD), jnp.float32),
    mesh=_sc_mesh,
    scratch_types=[
        pltpu.VMEM((NCH_DEG, CHUNK), jnp.int32),
        pltpu.VMEM((NPD,), jnp.float32),
        pltpu.VMEM((NS, NPDT), jnp.float32),
        pltpu.VMEM((NPDT,), jnp.float32),
        pltpu.VMEM_SHARED((NS, NPD), jnp.float32),
    ],
    compiler_params=_sc_params,
)
def _deg_sc(dst_hbm, deg_hbm, idx_v, hist_v, red_v, ob_v, sh_v):
    c = lax.axis_index("c")
    s = lax.axis_index("s")
    wid = s * NC + c
    pltpu.sync_copy(dst_hbm.at[wid], idx_v)

    @pl.loop(0, NPD, step=L)
    def _(i):
        hist_v[pl.ds(i, L)] = jnp.zeros((L,), jnp.float32)

    ones = jnp.ones((L,), jnp.float32)

    @pl.loop(0, NCH_DEG)
    def _(i):
        @pl.loop(0, CHUNK, step=L)
        def _(j):
            v = idx_v[i, pl.ds(j, L)]
            plsc.addupdate_scatter(hist_v, [v], ones)

    pltpu.sync_copy(hist_v, sh_v.at[s])
    plsc.subcore_barrier()

    col0 = s * NPDT
    for r in range(NS):
        pltpu.sync_copy(sh_v.at[r, pl.ds(col0, NPDT)], red_v.at[r])

    @pl.loop(0, NPDT, step=L)
    def _(j):
        acc = red_v[0, pl.ds(j, L)]
        for r in range(1, NS):
            acc = acc + red_v[r, pl.ds(j, L)]
        ob_v[pl.ds(j, L)] = acc

    pltpu.sync_copy(ob_v, deg_hbm.at[c, pl.ds(col0, NPDT)])


NBR = 8
NBI = 8
UNROLL = 8


@functools.partial(
    pl.kernel,
    out_type=jax.ShapeDtypeStruct((NC, NPH, D), jnp.float32),
    mesh=_sc_mesh,
    scratch_types=[
        pltpu.VMEM((NBI, 2, CHUNK_A), jnp.int32),
        pltpu.VMEM((NBR, CHUNK_A, D), jnp.float32),
        pltpu.SemaphoreType.DMA((NBI,)),
        pltpu.SemaphoreType.DMA((NBR,)),
        pltpu.SemaphoreType.DMA((NBR,)),
        pltpu.VMEM_SHARED((NPHA, D), jnp.float32),
    ],
    compiler_params=_sc_params,
)
def _agg_sc(edges_hbm, data_hbm, out_hbm,
            idx_v, rows_v, isem, gsem, ssem, acc_sh):
    c = lax.axis_index("c")
    s = lax.axis_index("s")

    zu = min(ZROWS, CHUNK_A)

    @pl.loop(0, zu)
    def _(r):
        @pl.loop(0, D, step=L)
        def _(j):
            rows_v[0, r, pl.ds(j, L)] = jnp.zeros((L,), jnp.float32)

    row0 = s * WROWS
    for b in range(WROWS // zu):
        pltpu.sync_copy(rows_v.at[0, pl.ds(0, zu)],
                        acc_sh.at[pl.ds(row0 + b * zu, zu)])
    plsc.subcore_barrier()

    base = c * NPH
    trash = jnp.full((L,), NPH, jnp.int32) + lax.iota(jnp.int32, L)

    def start_idx(i, bi):
        pltpu.async_copy(edges_hbm.at[s, i], idx_v.at[bi], isem.at[bi])

    def wait_idx(i, bi):
        pltpu.make_async_copy(edges_hbm.at[s, i], idx_v.at[bi],
                              isem.at[bi]).wait()

    def translate(bi):
        for j in range(0, CHUNK_A, L):
            d = idx_v[bi, 1, pl.ds(j, L)]
            local = d - base
            inb = (local >= 0) & (local < NPH)
            idx_v[bi, 1, pl.ds(j, L)] = jnp.where(inb, local, trash)

    def start_g(br, bi):
        pltpu.async_copy(data_hbm.at[idx_v.at[bi, 0]], rows_v.at[br],
                         gsem.at[br])

    def wait_g(br, bi):
        pltpu.make_async_copy(data_hbm.at[idx_v.at[bi, 0]], rows_v.at[br],
                              gsem.at[br]).wait()

    def start_s(br, bi):
        pltpu.async_copy(rows_v.at[br], acc_sh.at[idx_v.at[bi, 1]],
                         ssem.at[br], add=True)

    def wait_s(br, bi):
        pltpu.make_async_copy(rows_v.at[br], acc_sh.at[idx_v.at[bi, 1]],
                              ssem.at[br]).wait()

    def slot(j, r, head=False):
        if not isinstance(j, int) or j + 2 < NCH_AGG:
            b2r, b2i = (r + 2) % NBR, (r + 2) % NBI
            wait_idx(j + 2, b2i)
            translate(b2i)
            if not head:
                wait_s(b2r, b2i)
            start_g(b2r, b2i)
        wait_g(r % NBR, r % NBI)
        start_s(r % NBR, r % NBI)
        if isinstance(j, int):
            if j + 6 < NCH_AGG:
                start_idx(j + 6, (r + 6) % NBI)
        else:
            start_idx(j + 6, (r + 6) % NBI)

    for i in range(6):
        start_idx(i, i)
    wait_idx(0, 0)
    translate(0)
    wait_idx(1, 1)
    translate(1)
    start_g(0, 0)
    start_g(1, 1)

    for j in range(UNROLL):
        slot(j, j, head=(j < NBR - 2))

    tail_start = ((NCH_AGG - 7) // UNROLL) * UNROLL

    @pl.loop(1, tail_start // UNROLL)
    def _(k):
        for r in range(UNROLL):
            slot(k * UNROLL + r, r)

    for j in range(tail_start, NCH_AGG):
        slot(j, j % UNROLL)

    for ch in range(NCH_AGG - NBR, NCH_AGG):
        wait_s(ch % NBR, ch % NBI)

    plsc.subcore_barrier()
    pltpu.sync_copy(acc_sh.at[pl.ds(row0, WROWS)], out_hbm.at[c, pl.ds(row0, WROWS)])


BN = 400
GRID = N // BN


def _pre_body(deg_ref, x_ref, xs_ref, dis_ref):
    deg = deg_ref[0] + deg_ref[1] + 1.0
    dis = lax.rsqrt(deg)
    dis_ref[...] = dis
    xs_ref[...] = x_ref[...] * dis


_pre_tc = pl.pallas_call(
    _pre_body,
    grid=(GRID,),
    in_specs=[
        pl.BlockSpec((2, BN, 1), lambda i: (0, i, 0)),
        pl.BlockSpec((BN, D), lambda i: (i, 0)),
    ],
    out_specs=[
        pl.BlockSpec((BN, D), lambda i: (i, 0)),
        pl.BlockSpec((BN, 1), lambda i: (i, 0)),
    ],
    out_shape=[
        jax.ShapeDtypeStruct((N, D), jnp.float32),
        jax.ShapeDtypeStruct((N, 1), jnp.float32),
    ],
)


def _mm_body(p_ref, xs_ref, dis_ref, w1_ref, b1_ref, w2_ref, t_ref):
    dis = dis_ref[...]
    z = (p_ref[...] + xs_ref[...]) * dis
    h = jnp.dot(z, w1_ref[...], preferred_element_type=jnp.float32) + b1_ref[...]
    h = jnp.maximum(h, 0.0)
    t_ref[...] = jnp.dot(h, w2_ref[...], preferred_element_type=jnp.float32) * dis


_mm_tc = pl.pallas_call(
    _mm_body,
    grid=(GRID,),
    in_specs=[
        pl.BlockSpec((BN, D), lambda i: (i, 0)),
        pl.BlockSpec((BN, D), lambda i: (i, 0)),
        pl.BlockSpec((BN, 1), lambda i: (i, 0)),
        pl.BlockSpec((D, 2 * D), lambda i: (0, 0)),
        pl.BlockSpec((1, 2 * D), lambda i: (0, 0)),
        pl.BlockSpec((2 * D, D), lambda i: (0, 0)),
    ],
    out_specs=pl.BlockSpec((BN, D), lambda i: (i, 0)),
    out_shape=jax.ShapeDtypeStruct((N, D), jnp.float32),
)


def _out_body(p_ref, t_ref, dis_ref, b2_ref, o_ref):
    o_ref[...] = (p_ref[...] + t_ref[...]) * dis_ref[...] + b2_ref[...]


_out_tc = pl.pallas_call(
    _out_body,
    grid=(GRID,),
    in_specs=[
        pl.BlockSpec((BN, D), lambda i: (i, 0)),
        pl.BlockSpec((BN, D), lambda i: (i, 0)),
        pl.BlockSpec((BN, 1), lambda i: (i, 0)),
        pl.BlockSpec((1, D), lambda i: (0, 0)),
    ],
    out_specs=pl.BlockSpec((BN, D), lambda i: (i, 0)),
    out_shape=jax.ShapeDtypeStruct((N, D), jnp.float32),
)


def kernel(x, edge_index, conv1_weight, conv1_bias, conv2_weight, conv2_bias):
    dst = edge_index[1]
    dst_deg3 = dst.reshape(NW, NCH_DEG, CHUNK)
    ept = E // NS
    epa = NCH_AGG * CHUNK_A
    pad = ((0, 0), (0, epa - ept))
    srcp = jnp.pad(edge_index[0].reshape(NS, ept), pad)
    dstp = jnp.pad(dst.reshape(NS, ept), pad, constant_values=N)
    edges3 = jnp.stack([srcp.reshape(NS, NCH_AGG, CHUNK_A),
                        dstp.reshape(NS, NCH_AGG, CHUNK_A)], axis=2)

    deg_parts = _deg_sc(dst_deg3)
    deg2 = deg_parts[:, :N].reshape(2, N, 1)
    xs, dis = _pre_tc(deg2, x)
    p1 = _agg_sc(edges3, xs).reshape(NC * NPH, D)
    t = _mm_tc(p1, xs, dis, conv1_weight,
               conv1_bias.reshape(1, 2 * D), conv2_weight)
    p2 = _agg_sc(edges3, t).reshape(NC * NPH, D)
    out = _out_tc(p2, t, dis, conv2_bias.reshape(1, D))
    return out


# speedup vs baseline: 1.4763x; 1.2061x over previous
import functools

import jax
import jax.numpy as jnp
from jax import lax
from jax.experimental import pallas as pl
from jax.experimental.pallas import tpu as pltpu
from jax.experimental.pallas import tpu_sc as plsc

N = 10000
E = 320000
D = 128
NC = 2
NS = 16
L = 16
NW = NC * NS
CHUNK = 80
NCH_DEG = E // NW // CHUNK
CHUNK_A = 80
NCH_AGG = 250

NPD = 10240
NPDT = NPD // NS

NPH = 5120
NPHA = 5248
WROWS = NPH // NS
ZROWS = 80

_sc_mesh = plsc.VectorSubcoreMesh(core_axis_name="c", subcore_axis_name="s")
_sc_params = pltpu.CompilerParams(needs_layout_passes=False)


@functools.partial(
    pl.kernel,
    out_type=jax.ShapeDtypeStruct((NC, NPD), jnp.float32),
    mesh=_sc_mesh,
    scratch_types=[
        pltpu.VMEM((NCH_DEG, CHUNK), jnp.int32),
        pltpu.VMEM((NPD,), jnp.float32),
        pltpu.VMEM((NS, NPDT), jnp.float32),
        pltpu.VMEM((NPDT,), jnp.float32),
        pltpu.VMEM_SHARED((NS, NPD), jnp.float32),
    ],
    compiler_params=_sc_params,
)
def _deg_sc(dst_hbm, deg_hbm, idx_v, hist_v, red_v, ob_v, sh_v):
    c = lax.axis_index("c")
    s = lax.axis_index("s")
    wid = s * NC + c
    pltpu.sync_copy(dst_hbm.at[wid], idx_v)

    @pl.loop(0, NPD, step=L)
    def _(i):
        hist_v[pl.ds(i, L)] = jnp.zeros((L,), jnp.float32)

    ones = jnp.ones((L,), jnp.float32)

    @pl.loop(0, NCH_DEG)
    def _(i):
        @pl.loop(0, CHUNK, step=L)
        def _(j):
            v = idx_v[i, pl.ds(j, L)]
            plsc.addupdate_scatter(hist_v, [v], ones)

    pltpu.sync_copy(hist_v, sh_v.at[s])
    plsc.subcore_barrier()

    col0 = s * NPDT
    for r in range(NS):
        pltpu.sync_copy(sh_v.at[r, pl.ds(col0, NPDT)], red_v.at[r])

    @pl.loop(0, NPDT, step=L)
    def _(j):
        acc = red_v[0, pl.ds(j, L)]
        for r in range(1, NS):
            acc = acc + red_v[r, pl.ds(j, L)]
        ob_v[pl.ds(j, L)] = acc

    pltpu.sync_copy(ob_v, deg_hbm.at[c, pl.ds(col0, NPDT)])


NBR = 4
NBI = 8
UNROLL = 8


@functools.partial(
    pl.kernel,
    out_type=jax.ShapeDtypeStruct((NC, NPH, D), jnp.float32),
    mesh=_sc_mesh,
    scratch_types=[
        pltpu.VMEM((NBI, 2, CHUNK_A), jnp.int32),
        pltpu.VMEM((NBR, CHUNK_A, D), jnp.float32),
        pltpu.SemaphoreType.DMA((NBI,)),
        pltpu.SemaphoreType.DMA((NBR,)),
        pltpu.SemaphoreType.DMA((NBR,)),
        pltpu.VMEM_SHARED((NPHA, D), jnp.float32),
    ],
    compiler_params=_sc_params,
)
def _agg_sc(edges_hbm, data_hbm, out_hbm,
            idx_v, rows_v, isem, gsem, ssem, acc_sh):
    c = lax.axis_index("c")
    s = lax.axis_index("s")

    zu = min(ZROWS, CHUNK_A)

    @pl.loop(0, zu)
    def _(r):
        @pl.loop(0, D, step=L)
        def _(j):
            rows_v[0, r, pl.ds(j, L)] = jnp.zeros((L,), jnp.float32)

    row0 = s * WROWS
    for b in range(WROWS // zu):
        pltpu.sync_copy(rows_v.at[0, pl.ds(0, zu)],
                        acc_sh.at[pl.ds(row0 + b * zu, zu)])
    plsc.subcore_barrier()

    base = c * NPH
    trash = jnp.full((L,), NPH, jnp.int32) + lax.iota(jnp.int32, L)

    def start_idx(i, bi):
        pltpu.async_copy(edges_hbm.at[s, i], idx_v.at[bi], isem.at[bi])

    def wait_idx(i, bi):
        pltpu.make_async_copy(edges_hbm.at[s, i], idx_v.at[bi],
                              isem.at[bi]).wait()

    def translate(bi):
        for j in range(0, CHUNK_A, L):
            d = idx_v[bi, 1, pl.ds(j, L)]
            local = d - base
            inb = (local >= 0) & (local < NPH)
            idx_v[bi, 1, pl.ds(j, L)] = jnp.where(inb, local, trash)

    def start_g(br, bi):
        pltpu.async_copy(data_hbm.at[idx_v.at[bi, 0]], rows_v.at[br],
                         gsem.at[br])

    def wait_g(br, bi):
        pltpu.make_async_copy(data_hbm.at[idx_v.at[bi, 0]], rows_v.at[br],
                              gsem.at[br]).wait()

    def start_s(br, bi):
        pltpu.async_copy(rows_v.at[br], acc_sh.at[idx_v.at[bi, 1]],
                         ssem.at[br], add=True)

    def wait_s(br, bi):
        pltpu.make_async_copy(rows_v.at[br], acc_sh.at[idx_v.at[bi, 1]],
                              ssem.at[br]).wait()

    def slot(j, r, head=False):
        if not isinstance(j, int) or j + 2 < NCH_AGG:
            b2r, b2i = (r + 2) % NBR, (r + 2) % NBI
            wait_idx(j + 2, b2i)
            translate(b2i)
            if not head:
                wait_s(b2r, b2i)
            start_g(b2r, b2i)
        wait_g(r % NBR, r % NBI)
        start_s(r % NBR, r % NBI)
        if isinstance(j, int):
            if j + 6 < NCH_AGG:
                start_idx(j + 6, (r + 6) % NBI)
        else:
            start_idx(j + 6, (r + 6) % NBI)

    for i in range(6):
        start_idx(i, i)
    wait_idx(0, 0)
    translate(0)
    wait_idx(1, 1)
    translate(1)
    start_g(0, 0)
    start_g(1, 1)

    for j in range(UNROLL):
        slot(j, j, head=(j < NBR - 2))

    tail_start = ((NCH_AGG - 7) // UNROLL) * UNROLL

    @pl.loop(1, tail_start // UNROLL)
    def _(k):
        for r in range(UNROLL):
            slot(k * UNROLL + r, r)

    for j in range(tail_start, NCH_AGG):
        slot(j, j % UNROLL)

    for ch in range(NCH_AGG - NBR, NCH_AGG):
        wait_s(ch % NBR, ch % NBI)

    plsc.subcore_barrier()
    pltpu.sync_copy(acc_sh.at[pl.ds(row0, WROWS)], out_hbm.at[c, pl.ds(row0, WROWS)])


BN = 400
GRID = N // BN


def _pre_body(deg_ref, x_ref, xs_ref, dis_ref):
    deg = deg_ref[0] + deg_ref[1] + 1.0
    dis = lax.rsqrt(deg)
    dis_ref[...] = dis
    xs_ref[...] = x_ref[...] * dis


_pre_tc = pl.pallas_call(
    _pre_body,
    grid=(GRID,),
    in_specs=[
        pl.BlockSpec((2, BN, 1), lambda i: (0, i, 0)),
        pl.BlockSpec((BN, D), lambda i: (i, 0)),
    ],
    out_specs=[
        pl.BlockSpec((BN, D), lambda i: (i, 0)),
        pl.BlockSpec((BN, 1), lambda i: (i, 0)),
    ],
    out_shape=[
        jax.ShapeDtypeStruct((N, D), jnp.float32),
        jax.ShapeDtypeStruct((N, 1), jnp.float32),
    ],
)


def _mm_body(p_ref, xs_ref, dis_ref, w1_ref, b1_ref, w2_ref, t_ref):
    dis = dis_ref[...]
    z = (p_ref[...] + xs_ref[...]) * dis
    h = jnp.dot(z, w1_ref[...], preferred_element_type=jnp.float32) + b1_ref[...]
    h = jnp.maximum(h, 0.0)
    t_ref[...] = jnp.dot(h, w2_ref[...], preferred_element_type=jnp.float32) * dis


_mm_tc = pl.pallas_call(
    _mm_body,
    grid=(GRID,),
    in_specs=[
        pl.BlockSpec((BN, D), lambda i: (i, 0)),
        pl.BlockSpec((BN, D), lambda i: (i, 0)),
        pl.BlockSpec((BN, 1), lambda i: (i, 0)),
        pl.BlockSpec((D, 2 * D), lambda i: (0, 0)),
        pl.BlockSpec((1, 2 * D), lambda i: (0, 0)),
        pl.BlockSpec((2 * D, D), lambda i: (0, 0)),
    ],
    out_specs=pl.BlockSpec((BN, D), lambda i: (i, 0)),
    out_shape=jax.ShapeDtypeStruct((N, D), jnp.float32),
)


def _out_body(p_ref, t_ref, dis_ref, b2_ref, o_ref):
    o_ref[...] = (p_ref[...] + t_ref[...]) * dis_ref[...] + b2_ref[...]


_out_tc = pl.pallas_call(
    _out_body,
    grid=(GRID,),
    in_specs=[
        pl.BlockSpec((BN, D), lambda i: (i, 0)),
        pl.BlockSpec((BN, D), lambda i: (i, 0)),
        pl.BlockSpec((BN, 1), lambda i: (i, 0)),
        pl.BlockSpec((1, D), lambda i: (0, 0)),
    ],
    out_specs=pl.BlockSpec((BN, D), lambda i: (i, 0)),
    out_shape=jax.ShapeDtypeStruct((N, D), jnp.float32),
)


def kernel(x, edge_index, conv1_weight, conv1_bias, conv2_weight, conv2_bias):
    dst = edge_index[1]
    dst_deg3 = dst.reshape(NW, NCH_DEG, CHUNK)
    ept = E // NS
    epa = NCH_AGG * CHUNK_A
    pad = ((0, 0), (0, epa - ept))
    srcp = jnp.pad(edge_index[0].reshape(NS, ept), pad)
    dstp = jnp.pad(dst.reshape(NS, ept), pad, constant_values=N)
    edges3 = jnp.stack([srcp.reshape(NS, NCH_AGG, CHUNK_A),
                        dstp.reshape(NS, NCH_AGG, CHUNK_A)], axis=2)

    deg_parts = _deg_sc(dst_deg3)
    deg2 = deg_parts[:, :N].reshape(2, N, 1)
    xs, dis = _pre_tc(deg2, x)
    p1 = _agg_sc(edges3, xs).reshape(NC * NPH, D)
    t = _mm_tc(p1, xs, dis, conv1_weight,
               conv1_bias.reshape(1, 2 * D), conv2_weight)
    p2 = _agg_sc(edges3, t).reshape(NC * NPH, D)
    out = _out_tc(p2, t, dis, conv2_bias.reshape(1, D))
    return out
